# Initial kernel scaffold; baseline (speedup 1.0000x reference)
import jax
import jax.numpy as jnp
from jax.experimental import pallas as pl


def kernel(feats, edge_index, W, attn_l, attn_r, bias):
    raise NotImplementedError("write your pallas kernel here")


# speedup vs baseline: 72.0126x; 72.0126x over previous
import functools

import jax
import jax.numpy as jnp
from jax import lax
from jax.experimental import pallas as pl
from jax.experimental.pallas import tpu as pltpu
from jax.experimental.pallas import tpu_sc as plsc

N = 10000
E = 320000
IN = 128
H = 4
OUT = 32
NEG_SLOPE = 0.2

HD = 144
ED = 16
NC = 2
NS = 16
NW = NC * NS
EPW = E // NW
C = 80
NCHUNK = EPW // C
NPAD = 10240
RPT = NPAD // NS

BT = 1000
GRID = N // BT
BTF = 2048
GRIDF = 5


def _prep_body(x_ref, w_ref, alr_ref, hext_ref, er_ref):
    x = x_ref[...]
    h = jnp.dot(x, w_ref[...], preferred_element_type=jnp.float32)
    elr = jnp.dot(h, alr_ref[...], preferred_element_type=jnp.float32)
    z12 = jnp.zeros((BT, 12), jnp.float32)
    hext_ref[...] = jnp.concatenate([h, elr[:, 0:4], z12], axis=1)
    er_ref[...] = jnp.concatenate([elr[:, 4:8], z12], axis=1)


def _finish_body(n0_ref, n1_ref, d0_ref, d1_ref, x_ref, b_ref, o_ref):
    nsum = n0_ref[:, 0:128] + n1_ref[:, 0:128]
    dsum = d0_ref[...] + d1_ref[...]
    inv = 1.0 / jnp.maximum(dsum, 1e-9)
    parts = [nsum[:, h * OUT:(h + 1) * OUT] * inv[:, h:h + 1] for h in range(H)]
    o_ref[...] = jnp.concatenate(parts, axis=1) + x_ref[...] + b_ref[...]


def _edge_body(hext_hbm, er_hbm, src_hbm, dst_hbm,
               num_hbm, den_hbm,
               hrows_v, er_v, ee_v, src_v, dst_v,
               num_sp, den_sp, sem):
    cid = lax.axis_index("c")
    sid = lax.axis_index("s")
    wid = cid * NS + sid
    iota = lax.iota(jnp.int32, 16)
    zero16 = jnp.zeros((16,), jnp.float32)

    def _zb(r, carry):
        for j in range(HD // 16):
            hrows_v[r, pl.ds(j * 16, 16)] = zero16
        ee_v[r, pl.ds(0, 16)] = zero16
        return carry
    lax.fori_loop(0, C, _zb, None)
    rb = sid * RPT
    for t in range(RPT // C):
        pltpu.sync_copy(hrows_v, num_sp.at[pl.ds(rb + t * C, C)])
        pltpu.sync_copy(ee_v, den_sp.at[pl.ds(rb + t * C, C)])
    plsc.subcore_barrier()

    def _chunk(i, carry):
        base = wid * EPW + i * C
        pltpu.sync_copy(src_hbm.at[pl.ds(base, C)], src_v)
        pltpu.sync_copy(dst_hbm.at[pl.ds(base, C)], dst_v)
        g1 = pltpu.async_copy(hext_hbm.at[src_v], hrows_v, sem)
        g2 = pltpu.async_copy(er_hbm.at[dst_v], er_v, sem)
        g1.wait()
        g2.wait()
        for k in range(C * H // 16):
            lane = k * 16 + iota
            row = lane >> 2
            col = lane & 3
            a = plsc.load_gather(hrows_v, [row, col + IN])
            b = plsc.load_gather(er_v, [row, col])
            x = a + b
            e = jnp.maximum(x, 0.0) + NEG_SLOPE * jnp.minimum(x, 0.0)
            plsc.store_scatter(ee_v, [row, col], jnp.exp(e))
        def _scale(c, c2):
            svec = ee_v[c, pl.ds(0, 16)]
            for h in range(H):
                s = svec[h]
                for j in range(2):
                    sl = pl.ds(h * OUT + j * 16, 16)
                    hrows_v[c, sl] = hrows_v[c, sl] * s
            return c2
        lax.fori_loop(0, C, _scale, None)
        a1 = pltpu.async_copy(hrows_v, num_sp.at[dst_v], sem, add=True)
        a2 = pltpu.async_copy(ee_v, den_sp.at[dst_v], sem, add=True)
        a1.wait()
        a2.wait()
        return carry
    lax.fori_loop(0, NCHUNK, _chunk, None)
    plsc.subcore_barrier()

    for t in range(RPT // C):
        ro = rb + t * C
        go = cid * NPAD + ro
        pltpu.sync_copy(num_sp.at[pl.ds(ro, C)], hrows_v)
        pltpu.sync_copy(hrows_v, num_hbm.at[pl.ds(go, C)])
        pltpu.sync_copy(den_sp.at[pl.ds(ro, C)], ee_v)
        pltpu.sync_copy(ee_v, den_hbm.at[pl.ds(go, C)])


def kernel(feats, edge_index, W, attn_l, attn_r, bias):
    f32 = jnp.float32
    rows = jnp.arange(IN, dtype=jnp.int32)
    Al = jnp.zeros((IN, H), f32).at[rows, rows // OUT].set(attn_l.reshape(-1))
    Ar = jnp.zeros((IN, H), f32).at[rows, rows // OUT].set(attn_r.reshape(-1))
    Alr = jnp.concatenate([Al, Ar], axis=1)

    hext, er_t = pl.pallas_call(
        _prep_body,
        grid=(GRID,),
        in_specs=[
            pl.BlockSpec((BT, IN), lambda i: (i, 0)),
            pl.BlockSpec((IN, IN), lambda i: (0, 0)),
            pl.BlockSpec((IN, 2 * H), lambda i: (0, 0)),
        ],
        out_specs=[
            pl.BlockSpec((BT, HD), lambda i: (i, 0)),
            pl.BlockSpec((BT, ED), lambda i: (i, 0)),
        ],
        out_shape=[
            jax.ShapeDtypeStruct((N, HD), f32),
            jax.ShapeDtypeStruct((N, ED), f32),
        ],
    )(feats, W, Alr)

    src = edge_index[0]
    dst = edge_index[1]

    edge_kernel = pl.kernel(
        _edge_body,
        out_type=[
            jax.ShapeDtypeStruct((2 * NPAD, HD), f32),
            jax.ShapeDtypeStruct((2 * NPAD, ED), f32),
        ],
        mesh=plsc.VectorSubcoreMesh(
            core_axis_name="c", subcore_axis_name="s",
            num_cores=NC, num_subcores=NS),
        compiler_params=pltpu.CompilerParams(use_tc_tiling_on_sc=False, needs_layout_passes=False),
        scratch_types=[
            pltpu.VMEM((C, HD), f32),
            pltpu.VMEM((C, ED), f32),
            pltpu.VMEM((C, ED), f32),
            pltpu.VMEM((C,), jnp.int32),
            pltpu.VMEM((C,), jnp.int32),
            pltpu.VMEM_SHARED((NPAD, HD), f32),
            pltpu.VMEM_SHARED((NPAD, ED), f32),
            pltpu.SemaphoreType.DMA,
        ],
    )
    num_all, den_all = edge_kernel(hext, er_t, src, dst)

    half = NPAD // BTF
    out = pl.pallas_call(
        _finish_body,
        grid=(GRIDF,),
        in_specs=[
            pl.BlockSpec((BTF, HD), lambda i: (i, 0)),
            pl.BlockSpec((BTF, HD), lambda i: (i + half, 0)),
            pl.BlockSpec((BTF, ED), lambda i: (i, 0)),
            pl.BlockSpec((BTF, ED), lambda i: (i + half, 0)),
            pl.BlockSpec((BTF, IN), lambda i: (i, 0)),
            pl.BlockSpec((1, IN), lambda i: (0, 0)),
        ],
        out_specs=pl.BlockSpec((BTF, IN), lambda i: (i, 0)),
        out_shape=jax.ShapeDtypeStruct((N, IN), f32),
    )(num_all, num_all, den_all, den_all, feats, bias.reshape(1, IN))
    return out


# speedup vs baseline: 104.6255x; 1.4529x over previous
import jax
import jax.numpy as jnp
from jax import lax
from jax.experimental import pallas as pl
from jax.experimental.pallas import tpu as pltpu
from jax.experimental.pallas import tpu_sc as plsc

N = 10000
E = 320000
IN = 128
H = 4
OUT = 32
NEG_SLOPE = 0.2

ED = 16
NC = 2
NS = 16
NW = NC * NS
EPW = E // NW
C = 80
NCHUNK = EPW // C
NPAD = 10240
RPT = NPAD // NS

BT = 1000
GRID = N // BT
BTF = 2048
GRIDF = 5


def _prep_body(x_ref, w_ref, alr_ref, h_ref, lr_ref):
    x = x_ref[...]
    h = jnp.dot(x, w_ref[...], preferred_element_type=jnp.float32)
    elr = jnp.dot(h, alr_ref[...], preferred_element_type=jnp.float32)
    h_ref[...] = h
    lr_ref[...] = jnp.concatenate([elr, jnp.zeros((BT, 8), jnp.float32)], axis=1)


def _finish_body(n0_ref, n1_ref, d0_ref, d1_ref, x_ref, b_ref, o_ref):
    nsum = n0_ref[...] + n1_ref[...]
    dsum = d0_ref[...] + d1_ref[...]
    inv = 1.0 / jnp.maximum(dsum, 1e-9)
    parts = [nsum[:, h * OUT:(h + 1) * OUT] * inv[:, h:h + 1] for h in range(H)]
    o_ref[...] = jnp.concatenate(parts, axis=1) + x_ref[...] + b_ref[...]


def _edge_body(h_hbm, lr_hbm, src2_hbm, dst2_hbm,
               num_hbm, den_hbm,
               hrA, hrB, lsA, lsB, ldA, ldB, eeA, eeB, srcv, dstv,
               num_sp, den_sp, gsemA, gsemB, asem):
    cid = lax.axis_index("c")
    sid = lax.axis_index("s")
    wid = cid * NS + sid
    iota = lax.iota(jnp.int32, 16)
    zero16 = jnp.zeros((16,), jnp.float32)

    def _zb(r, carry):
        for j in range(IN // 16):
            hrA[r, pl.ds(j * 16, 16)] = zero16
        eeA[r, pl.ds(0, 16)] = zero16
        eeB[r, pl.ds(0, 16)] = zero16
        return carry
    lax.fori_loop(0, C, _zb, None)
    rb = sid * RPT
    for t in range(RPT // C):
        pltpu.sync_copy(hrA, num_sp.at[pl.ds(rb + t * C, C)])
        pltpu.sync_copy(eeA, den_sp.at[pl.ds(rb + t * C, C)])
    plsc.subcore_barrier()

    cb = wid * NCHUNK

    def _load_idx(k, b):
        pltpu.sync_copy(src2_hbm.at[pl.ds(cb + k, 1)], srcv.at[pl.ds(b, 1)])
        pltpu.sync_copy(dst2_hbm.at[pl.ds(cb + k, 1)], dstv.at[pl.ds(b, 1)])

    def _fire(b, hr, ls, ld, gsem):
        pltpu.async_copy(h_hbm.at[srcv.at[b]], hr, gsem)
        pltpu.async_copy(lr_hbm.at[srcv.at[b]], ls, gsem)
        pltpu.async_copy(lr_hbm.at[dstv.at[b]], ld, gsem)

    def _wait(b, hr, ls, ld, gsem):
        pltpu.make_async_copy(h_hbm.at[srcv.at[b]], hr, gsem).wait()
        pltpu.make_async_copy(lr_hbm.at[srcv.at[b]], ls, gsem).wait()
        pltpu.make_async_copy(lr_hbm.at[dstv.at[b]], ld, gsem).wait()

    def _compute(b, hr, ls, ld, ee):
        for kk in range(C * H // 16):
            lane = kk * 16 + iota
            row = lane >> 2
            col = lane & 3
            a = plsc.load_gather(ls, [row, col])
            bb = plsc.load_gather(ld, [row, col + 4])
            x = a + bb
            e = jnp.maximum(x, 0.0) + NEG_SLOPE * jnp.minimum(x, 0.0)
            plsc.store_scatter(ee, [row, col], jnp.exp(e))

        def _scale(c, c2):
            svec = ee[c, pl.ds(0, 16)]
            for h in range(H):
                sc = svec[h]
                for j in range(2):
                    sl = pl.ds(h * OUT + j * 16, 16)
                    hr[c, sl] = hr[c, sl] * sc
            return c2
        lax.fori_loop(0, C, _scale, None)

        a1 = pltpu.async_copy(hr, num_sp.at[dstv.at[b]], asem, add=True)
        a2 = pltpu.async_copy(ee, den_sp.at[dstv.at[b]], asem, add=True)
        a1.wait()
        a2.wait()

    _load_idx(0, 0)
    _load_idx(1, 1)
    _fire(0, hrA, lsA, ldA, gsemA)
    _fire(1, hrB, lsB, ldB, gsemB)

    def _chunk2(i, carry):
        k = 2 * i
        _wait(0, hrA, lsA, ldA, gsemA)
        _compute(0, hrA, lsA, ldA, eeA)
        _load_idx(k + 2, 0)
        _fire(0, hrA, lsA, ldA, gsemA)
        _wait(1, hrB, lsB, ldB, gsemB)
        _compute(1, hrB, lsB, ldB, eeB)

        @pl.when(k + 3 < NCHUNK)
        def _():
            _load_idx(k + 3, 1)
            _fire(1, hrB, lsB, ldB, gsemB)
        return carry
    lax.fori_loop(0, (NCHUNK - 1) // 2, _chunk2, None)
    _wait(0, hrA, lsA, ldA, gsemA)
    _compute(0, hrA, lsA, ldA, eeA)
    plsc.subcore_barrier()

    for t in range(RPT // C):
        ro = rb + t * C
        go = cid * NPAD + ro
        pltpu.sync_copy(num_sp.at[pl.ds(ro, C)], hrA)
        pltpu.sync_copy(hrA, num_hbm.at[pl.ds(go, C)])
        pltpu.sync_copy(den_sp.at[pl.ds(ro, C)], eeA)
        pltpu.sync_copy(eeA, den_hbm.at[pl.ds(go, C)])


def kernel(feats, edge_index, W, attn_l, attn_r, bias):
    f32 = jnp.float32
    rows = jnp.arange(IN, dtype=jnp.int32)
    Al = jnp.zeros((IN, H), f32).at[rows, rows // OUT].set(attn_l.reshape(-1))
    Ar = jnp.zeros((IN, H), f32).at[rows, rows // OUT].set(attn_r.reshape(-1))
    Alr = jnp.concatenate([Al, Ar], axis=1)

    h_t, lr_t = pl.pallas_call(
        _prep_body,
        grid=(GRID,),
        in_specs=[
            pl.BlockSpec((BT, IN), lambda i: (i, 0)),
            pl.BlockSpec((IN, IN), lambda i: (0, 0)),
            pl.BlockSpec((IN, 2 * H), lambda i: (0, 0)),
        ],
        out_specs=[
            pl.BlockSpec((BT, IN), lambda i: (i, 0)),
            pl.BlockSpec((BT, ED), lambda i: (i, 0)),
        ],
        out_shape=[
            jax.ShapeDtypeStruct((N, IN), f32),
            jax.ShapeDtypeStruct((N, ED), f32),
        ],
    )(feats, W, Alr)

    src2 = edge_index[0].reshape(E // C, C)
    dst2 = edge_index[1].reshape(E // C, C)

    edge_kernel = pl.kernel(
        _edge_body,
        out_type=[
            jax.ShapeDtypeStruct((2 * NPAD, IN), f32),
            jax.ShapeDtypeStruct((2 * NPAD, ED), f32),
        ],
        mesh=plsc.VectorSubcoreMesh(
            core_axis_name="c", subcore_axis_name="s",
            num_cores=NC, num_subcores=NS),
        compiler_params=pltpu.CompilerParams(
            use_tc_tiling_on_sc=False, needs_layout_passes=False),
        scratch_types=[
            pltpu.VMEM((C, IN), f32),
            pltpu.VMEM((C, IN), f32),
            pltpu.VMEM((C, ED), f32),
            pltpu.VMEM((C, ED), f32),
            pltpu.VMEM((C, ED), f32),
            pltpu.VMEM((C, ED), f32),
            pltpu.VMEM((C, ED), f32),
            pltpu.VMEM((C, ED), f32),
            pltpu.VMEM((2, C), jnp.int32),
            pltpu.VMEM((2, C), jnp.int32),
            pltpu.VMEM_SHARED((NPAD, IN), f32),
            pltpu.VMEM_SHARED((NPAD, ED), f32),
            pltpu.SemaphoreType.DMA,
            pltpu.SemaphoreType.DMA,
            pltpu.SemaphoreType.DMA,
        ],
    )
    num_all, den_all = edge_kernel(h_t, lr_t, src2, dst2)

    half = NPAD // BTF
    out = pl.pallas_call(
        _finish_body,
        grid=(GRIDF,),
        in_specs=[
            pl.BlockSpec((BTF, IN), lambda i: (i, 0)),
            pl.BlockSpec((BTF, IN), lambda i: (i + half, 0)),
            pl.BlockSpec((BTF, ED), lambda i: (i, 0)),
            pl.BlockSpec((BTF, ED), lambda i: (i + half, 0)),
            pl.BlockSpec((BTF, IN), lambda i: (i, 0)),
            pl.BlockSpec((1, IN), lambda i: (0, 0)),
        ],
        out_specs=pl.BlockSpec((BTF, IN), lambda i: (i, 0)),
        out_shape=jax.ShapeDtypeStruct((N, IN), f32),
    )(num_all, num_all, den_all, den_all, feats, bias.reshape(1, IN))
    return out


# speedup vs baseline: 104.6824x; 1.0005x over previous
import jax
import jax.numpy as jnp
from jax import lax
from jax.experimental import pallas as pl
from jax.experimental.pallas import tpu as pltpu
from jax.experimental.pallas import tpu_sc as plsc

N = 10000
E = 320000
IN = 128
H = 4
OUT = 32
NEG_SLOPE = 0.2

HD = 144
ED = 16
NC = 2
NS = 16
NW = NC * NS
EPW = E // NW
C = 80
NCHUNK = EPW // C
NPAD = 10240
RPT = NPAD // NS

BT = 1000
GRID = N // BT
BTF = 2048
GRIDF = 5


def _prep_body(x_ref, w_ref, alr_ref, h_ref, lr_ref):
    x = x_ref[...]
    h = jnp.dot(x, w_ref[...], preferred_element_type=jnp.float32)
    elr = jnp.dot(h, alr_ref[...], preferred_element_type=jnp.float32)
    z12 = jnp.zeros((BT, 12), jnp.float32)
    h_ref[...] = jnp.concatenate([h, elr[:, 0:4], z12], axis=1)
    lr_ref[...] = jnp.concatenate([elr[:, 4:8], z12], axis=1)


def _finish_body(n0_ref, n1_ref, d0_ref, d1_ref, x_ref, b_ref, o_ref):
    nsum = n0_ref[:, 0:IN] + n1_ref[:, 0:IN]
    dsum = d0_ref[...] + d1_ref[...]
    inv = 1.0 / jnp.maximum(dsum, 1e-9)
    parts = [nsum[:, h * OUT:(h + 1) * OUT] * inv[:, h:h + 1] for h in range(H)]
    o_ref[...] = jnp.concatenate(parts, axis=1) + x_ref[...] + b_ref[...]


def _edge_body(h_hbm, lr_hbm, idx3_hbm,
               num_hbm, den_hbm,
               hrA, hrB, ldA, ldB, eeA, eeB, idxv,
               num_sp, den_sp, gsemA, gsemB, asem):
    cid = lax.axis_index("c")
    sid = lax.axis_index("s")
    wid = cid * NS + sid
    iota = lax.iota(jnp.int32, 16)
    zero16 = jnp.zeros((16,), jnp.float32)

    def _zb(r, carry):
        for j in range(HD // 16):
            hrA[r, pl.ds(j * 16, 16)] = zero16
        eeA[r, pl.ds(0, 16)] = zero16
        eeB[r, pl.ds(0, 16)] = zero16
        return carry
    lax.fori_loop(0, C, _zb, None)
    rb = sid * RPT
    for t in range(RPT // C):
        pltpu.sync_copy(hrA, num_sp.at[pl.ds(rb + t * C, C)])
        pltpu.sync_copy(eeA, den_sp.at[pl.ds(rb + t * C, C)])
    plsc.subcore_barrier()

    cb = wid * NCHUNK

    def _load_idx(k, b):
        pltpu.sync_copy(idx3_hbm.at[pl.ds(cb + k, 1)], idxv.at[pl.ds(b, 1)])

    def _fire(b, hr, ld, gsem):
        pltpu.async_copy(h_hbm.at[idxv.at[b, 0]], hr, gsem)
        pltpu.async_copy(lr_hbm.at[idxv.at[b, 1]], ld, gsem)

    def _wait(b, hr, ld, gsem):
        pltpu.make_async_copy(h_hbm.at[idxv.at[b, 0]], hr, gsem).wait()
        pltpu.make_async_copy(lr_hbm.at[idxv.at[b, 1]], ld, gsem).wait()

    def _compute(b, hr, ld, ee):
        for kk in range(C * H // 16):
            lane = kk * 16 + iota
            row = lane >> 2
            col = lane & 3
            a = plsc.load_gather(hr, [row, col + IN])
            bb = plsc.load_gather(ld, [row, col])
            x = a + bb
            e = jnp.maximum(x, 0.0) + NEG_SLOPE * jnp.minimum(x, 0.0)
            plsc.store_scatter(ee, [row, col], jnp.exp(e))

        def _scale(c, c2):
            svec = ee[c, pl.ds(0, 16)]
            for h in range(H):
                sc = svec[h]
                for j in range(2):
                    sl = pl.ds(h * OUT + j * 16, 16)
                    hr[c, sl] = hr[c, sl] * sc
            return c2
        lax.fori_loop(0, C, _scale, None)

        a1 = pltpu.async_copy(hr, num_sp.at[idxv.at[b, 1]], asem, add=True)
        a2 = pltpu.async_copy(ee, den_sp.at[idxv.at[b, 1]], asem, add=True)
        a1.wait()
        a2.wait()

    _load_idx(0, 0)
    _load_idx(1, 1)
    _fire(0, hrA, ldA, gsemA)
    _fire(1, hrB, ldB, gsemB)

    def _chunk2(i, carry):
        k = 2 * i
        _wait(0, hrA, ldA, gsemA)
        _compute(0, hrA, ldA, eeA)
        _load_idx(k + 2, 0)
        _fire(0, hrA, ldA, gsemA)
        _wait(1, hrB, ldB, gsemB)
        _compute(1, hrB, ldB, eeB)

        @pl.when(k + 3 < NCHUNK)
        def _():
            _load_idx(k + 3, 1)
            _fire(1, hrB, ldB, gsemB)
        return carry
    lax.fori_loop(0, (NCHUNK - 1) // 2, _chunk2, None)
    _wait(0, hrA, ldA, gsemA)
    _compute(0, hrA, ldA, eeA)
    plsc.subcore_barrier()

    for t in range(RPT // C):
        ro = rb + t * C
        go = cid * NPAD + ro
        pltpu.sync_copy(num_sp.at[pl.ds(ro, C)], hrA)
        pltpu.sync_copy(hrA, num_hbm.at[pl.ds(go, C)])
        pltpu.sync_copy(den_sp.at[pl.ds(ro, C)], eeA)
        pltpu.sync_copy(eeA, den_hbm.at[pl.ds(go, C)])


def kernel(feats, edge_index, W, attn_l, attn_r, bias):
    f32 = jnp.float32
    rows = jnp.arange(IN, dtype=jnp.int32)
    Al = jnp.zeros((IN, H), f32).at[rows, rows // OUT].set(attn_l.reshape(-1))
    Ar = jnp.zeros((IN, H), f32).at[rows, rows // OUT].set(attn_r.reshape(-1))
    Alr = jnp.concatenate([Al, Ar], axis=1)

    h_t, lr_t = pl.pallas_call(
        _prep_body,
        grid=(GRID,),
        in_specs=[
            pl.BlockSpec((BT, IN), lambda i: (i, 0)),
            pl.BlockSpec((IN, IN), lambda i: (0, 0)),
            pl.BlockSpec((IN, 2 * H), lambda i: (0, 0)),
        ],
        out_specs=[
            pl.BlockSpec((BT, HD), lambda i: (i, 0)),
            pl.BlockSpec((BT, ED), lambda i: (i, 0)),
        ],
        out_shape=[
            jax.ShapeDtypeStruct((N, HD), f32),
            jax.ShapeDtypeStruct((N, ED), f32),
        ],
    )(feats, W, Alr)

    idx3 = jnp.stack(
        [edge_index[0].reshape(E // C, C), edge_index[1].reshape(E // C, C)],
        axis=1)

    edge_kernel = pl.kernel(
        _edge_body,
        out_type=[
            jax.ShapeDtypeStruct((2 * NPAD, HD), f32),
            jax.ShapeDtypeStruct((2 * NPAD, ED), f32),
        ],
        mesh=plsc.VectorSubcoreMesh(
            core_axis_name="c", subcore_axis_name="s",
            num_cores=NC, num_subcores=NS),
        compiler_params=pltpu.CompilerParams(
            use_tc_tiling_on_sc=False, needs_layout_passes=False),
        scratch_types=[
            pltpu.VMEM((C, HD), f32),
            pltpu.VMEM((C, HD), f32),
            pltpu.VMEM((C, ED), f32),
            pltpu.VMEM((C, ED), f32),
            pltpu.VMEM((C, ED), f32),
            pltpu.VMEM((C, ED), f32),
            pltpu.VMEM((2, 2, C), jnp.int32),
            pltpu.VMEM_SHARED((NPAD, HD), f32),
            pltpu.VMEM_SHARED((NPAD, ED), f32),
            pltpu.SemaphoreType.DMA,
            pltpu.SemaphoreType.DMA,
            pltpu.SemaphoreType.DMA,
        ],
    )
    num_all, den_all = edge_kernel(h_t, lr_t, idx3)

    half = NPAD // BTF
    out = pl.pallas_call(
        _finish_body,
        grid=(GRIDF,),
        in_specs=[
            pl.BlockSpec((BTF, HD), lambda i: (i, 0)),
            pl.BlockSpec((BTF, HD), lambda i: (i + half, 0)),
            pl.BlockSpec((BTF, ED), lambda i: (i, 0)),
            pl.BlockSpec((BTF, ED), lambda i: (i + half, 0)),
            pl.BlockSpec((BTF, IN), lambda i: (i, 0)),
            pl.BlockSpec((1, IN), lambda i: (0, 0)),
        ],
        out_specs=pl.BlockSpec((BTF, IN), lambda i: (i, 0)),
        out_shape=jax.ShapeDtypeStruct((N, IN), f32),
    )(num_all, num_all, den_all, den_all, feats, bias.reshape(1, IN))
    return out


# speedup vs baseline: 105.7667x; 1.0104x over previous
import jax
import jax.numpy as jnp
from jax import lax
from jax.experimental import pallas as pl
from jax.experimental.pallas import tpu as pltpu
from jax.experimental.pallas import tpu_sc as plsc

N = 10000
E = 320000
IN = 128
H = 4
OUT = 32
NEG_SLOPE = 0.2

HD = 144
ED = 16
NC = 2
NS = 16
NW = NC * NS
EPW = E // NW
C = 80
NCHUNK = EPW // C
NPAD = 10240
RPT = NPAD // NS

BT = 1000
GRID = N // BT
BTF = 2048
GRIDF = 5


def _prep_body(x_ref, w_ref, alr_ref, h_ref, lr_ref):
    x = x_ref[...]
    h = jnp.dot(x, w_ref[...], preferred_element_type=jnp.float32)
    elr = jnp.dot(h, alr_ref[...], preferred_element_type=jnp.float32)
    z12 = jnp.zeros((BT, 12), jnp.float32)
    h_ref[...] = jnp.concatenate([h, elr[:, 0:4], z12], axis=1)
    lr_ref[...] = jnp.concatenate([elr[:, 4:8], z12], axis=1)


def _finish_body(n0_ref, n1_ref, d0_ref, d1_ref, x_ref, b_ref, o_ref):
    nsum = n0_ref[:, 0:IN] + n1_ref[:, 0:IN]
    dsum = d0_ref[...] + d1_ref[...]
    inv = 1.0 / jnp.maximum(dsum, 1e-9)
    parts = [nsum[:, h * OUT:(h + 1) * OUT] * inv[:, h:h + 1] for h in range(H)]
    o_ref[...] = jnp.concatenate(parts, axis=1) + x_ref[...] + b_ref[...]


def _edge_body(h_hbm, lr_hbm, idx3_hbm,
               num_hbm, den_hbm,
               hrA, hrB, ldA, ldB, eeA, eeB, idxv,
               num_sp, den_sp, gsemA, gsemB, asem):
    cid = lax.axis_index("c")
    sid = lax.axis_index("s")
    wid = cid * NS + sid
    iota = lax.iota(jnp.int32, 16)
    zero16 = jnp.zeros((16,), jnp.float32)

    def _zb(r, carry):
        for j in range(HD // 16):
            hrA[r, pl.ds(j * 16, 16)] = zero16
        eeA[r, pl.ds(0, 16)] = zero16
        eeB[r, pl.ds(0, 16)] = zero16
        return carry
    lax.fori_loop(0, C, _zb, None)
    rb = sid * RPT
    for t in range(RPT // C):
        pltpu.sync_copy(hrA, num_sp.at[pl.ds(rb + t * C, C)])
        pltpu.sync_copy(eeA, den_sp.at[pl.ds(rb + t * C, C)])
    plsc.subcore_barrier()

    cb = wid * NCHUNK

    def _load_idx(k, b):
        pltpu.sync_copy(idx3_hbm.at[pl.ds(cb + k, 1)], idxv.at[pl.ds(b, 1)])

    def _fire(b, hr, ld, gsem):
        pltpu.async_copy(h_hbm.at[idxv.at[b, 0]], hr, gsem)
        pltpu.async_copy(lr_hbm.at[idxv.at[b, 1]], ld, gsem)

    def _wait(b, hr, ld, gsem):
        pltpu.make_async_copy(h_hbm.at[idxv.at[b, 0]], hr, gsem).wait()
        pltpu.make_async_copy(lr_hbm.at[idxv.at[b, 1]], ld, gsem).wait()

    def _compute(b, hr, ld, ee):
        for kk in range(C * H // 16):
            lane = kk * 16 + iota
            row = lane >> 2
            col = lane & 3
            a = plsc.load_gather(hr, [row, col + IN])
            bb = plsc.load_gather(ld, [row, col])
            x = a + bb
            e = jnp.maximum(x, 0.0) + NEG_SLOPE * jnp.minimum(x, 0.0)
            plsc.store_scatter(ee, [row, col], jnp.exp(e))

        def _scale(c, c2):
            svec = ee[c, pl.ds(0, 16)]
            for h in range(H):
                sc = svec[h]
                for j in range(2):
                    sl = pl.ds(h * OUT + j * 16, 16)
                    hr[c, sl] = hr[c, sl] * sc
            return c2
        lax.fori_loop(0, C // 4, lambda q, c2: [_scale(4 * q + r, c2) for r in range(4)][-1], None)

        a1 = pltpu.async_copy(hr, num_sp.at[idxv.at[b, 1]], asem, add=True)
        a2 = pltpu.async_copy(ee, den_sp.at[idxv.at[b, 1]], asem, add=True)
        a1.wait()
        a2.wait()

    _load_idx(0, 0)
    _load_idx(1, 1)
    _fire(0, hrA, ldA, gsemA)
    _fire(1, hrB, ldB, gsemB)

    def _chunk2(i, carry):
        k = 2 * i
        _wait(0, hrA, ldA, gsemA)
        _compute(0, hrA, ldA, eeA)
        _load_idx(k + 2, 0)
        _fire(0, hrA, ldA, gsemA)
        _wait(1, hrB, ldB, gsemB)
        _compute(1, hrB, ldB, eeB)

        @pl.when(k + 3 < NCHUNK)
        def _():
            _load_idx(k + 3, 1)
            _fire(1, hrB, ldB, gsemB)
        return carry
    lax.fori_loop(0, (NCHUNK - 1) // 2, _chunk2, None)
    _wait(0, hrA, ldA, gsemA)
    _compute(0, hrA, ldA, eeA)
    plsc.subcore_barrier()

    for t in range(RPT // C):
        ro = rb + t * C
        go = cid * NPAD + ro
        pltpu.sync_copy(num_sp.at[pl.ds(ro, C)], hrA)
        pltpu.sync_copy(hrA, num_hbm.at[pl.ds(go, C)])
        pltpu.sync_copy(den_sp.at[pl.ds(ro, C)], eeA)
        pltpu.sync_copy(eeA, den_hbm.at[pl.ds(go, C)])


def kernel(feats, edge_index, W, attn_l, attn_r, bias):
    f32 = jnp.float32
    rows = jnp.arange(IN, dtype=jnp.int32)
    Al = jnp.zeros((IN, H), f32).at[rows, rows // OUT].set(attn_l.reshape(-1))
    Ar = jnp.zeros((IN, H), f32).at[rows, rows // OUT].set(attn_r.reshape(-1))
    Alr = jnp.concatenate([Al, Ar], axis=1)

    h_t, lr_t = pl.pallas_call(
        _prep_body,
        grid=(GRID,),
        in_specs=[
            pl.BlockSpec((BT, IN), lambda i: (i, 0)),
            pl.BlockSpec((IN, IN), lambda i: (0, 0)),
            pl.BlockSpec((IN, 2 * H), lambda i: (0, 0)),
        ],
        out_specs=[
            pl.BlockSpec((BT, HD), lambda i: (i, 0)),
            pl.BlockSpec((BT, ED), lambda i: (i, 0)),
        ],
        out_shape=[
            jax.ShapeDtypeStruct((N, HD), f32),
            jax.ShapeDtypeStruct((N, ED), f32),
        ],
    )(feats, W, Alr)

    idx3 = jnp.stack(
        [edge_index[0].reshape(E // C, C), edge_index[1].reshape(E // C, C)],
        axis=1)

    edge_kernel = pl.kernel(
        _edge_body,
        out_type=[
            jax.ShapeDtypeStruct((2 * NPAD, HD), f32),
            jax.ShapeDtypeStruct((2 * NPAD, ED), f32),
        ],
        mesh=plsc.VectorSubcoreMesh(
            core_axis_name="c", subcore_axis_name="s",
            num_cores=NC, num_subcores=NS),
        compiler_params=pltpu.CompilerParams(
            use_tc_tiling_on_sc=False, needs_layout_passes=False),
        scratch_types=[
            pltpu.VMEM((C, HD), f32),
            pltpu.VMEM((C, HD), f32),
            pltpu.VMEM((C, ED), f32),
            pltpu.VMEM((C, ED), f32),
            pltpu.VMEM((C, ED), f32),
            pltpu.VMEM((C, ED), f32),
            pltpu.VMEM((2, 2, C), jnp.int32),
            pltpu.VMEM_SHARED((NPAD, HD), f32),
            pltpu.VMEM_SHARED((NPAD, ED), f32),
            pltpu.SemaphoreType.DMA,
            pltpu.SemaphoreType.DMA,
            pltpu.SemaphoreType.DMA,
        ],
    )
    num_all, den_all = edge_kernel(h_t, lr_t, idx3)

    half = NPAD // BTF
    out = pl.pallas_call(
        _finish_body,
        grid=(GRIDF,),
        in_specs=[
            pl.BlockSpec((BTF, HD), lambda i: (i, 0)),
            pl.BlockSpec((BTF, HD), lambda i: (i + half, 0)),
            pl.BlockSpec((BTF, ED), lambda i: (i, 0)),
            pl.BlockSpec((BTF, ED), lambda i: (i + half, 0)),
            pl.BlockSpec((BTF, IN), lambda i: (i, 0)),
            pl.BlockSpec((1, IN), lambda i: (0, 0)),
        ],
        out_specs=pl.BlockSpec((BTF, IN), lambda i: (i, 0)),
        out_shape=jax.ShapeDtypeStruct((N, IN), f32),
    )(num_all, num_all, den_all, den_all, feats, bias.reshape(1, IN))
    return out
